# baseline (device time: 34193 ns/iter reference)
import jax
import jax.numpy as jnp
from jax import lax
from jax.experimental import pallas as pl
from jax.experimental.pallas import tpu as pltpu

N_DEV = 4
N_LOCAL_EXPERTS = 4
N_TOKENS = 1024
D_MODEL = 256
D_HID = 512
CHUNK = N_TOKENS // N_DEV


def kernel(x, router_W, route_idx, expert_W, shared_W):
    def body(x_ref, router_ref, idx_ref, ew_ref, sw_ref, out_ref,
             partial_ref, comm_ref, send_sems, recv_sems):
        my = lax.axis_index("i")
        left = (my + N_DEV - 1) % N_DEV
        right = (my + 1) % N_DEV

        barrier_sem = pltpu.get_barrier_semaphore()
        for nbr in (left, right):
            pl.semaphore_signal(
                barrier_sem, inc=1,
                device_id=(nbr,), device_id_type=pl.DeviceIdType.MESH,
            )
        pl.semaphore_wait(barrier_sem, 2)

        xv = x_ref[:, :]
        scores = jnp.dot(xv, router_ref[:, :], preferred_element_type=jnp.float32)
        s_max = jnp.max(scores, axis=-1, keepdims=True)
        e_s = jnp.exp(scores - s_max)
        probs = e_s / jnp.sum(e_s, axis=-1, keepdims=True)
        idx = idx_ref[:, :]
        col = lax.broadcasted_iota(jnp.int32, scores.shape, 1)
        gate = jnp.sum(jnp.where(col == idx, probs, 0.0), axis=-1,
                       keepdims=True)

        partial = jnp.zeros((N_TOKENS, D_HID), dtype=jnp.float32)
        for e in range(N_LOCAL_EXPERTS):
            ge = my * N_LOCAL_EXPERTS + e
            w = jnp.where(idx == ge, gate, 0.0)
            contrib = jnp.dot(xv, ew_ref[e], preferred_element_type=jnp.float32)
            partial = partial + w * contrib
        partial_ref[:, :] = partial

        x_own = x_ref[pl.ds(my * CHUNK, CHUNK), :]
        shared = jnp.dot(x_own, sw_ref[:, :], preferred_element_type=jnp.float32)

        first_chunk = (my + N_DEV - 1) % N_DEV
        comm_ref[0, :, :] = partial_ref[pl.ds(first_chunk * CHUNK, CHUNK), :]

        for s in range(N_DEV - 1):
            rdma = pltpu.make_async_remote_copy(
                src_ref=comm_ref.at[s],
                dst_ref=comm_ref.at[s + 1],
                send_sem=send_sems.at[s],
                recv_sem=recv_sems.at[s],
                device_id=(right,),
                device_id_type=pl.DeviceIdType.MESH,
            )
            rdma.start()
            rdma.wait()

            c = (my + ((2 - s) % N_DEV)) % N_DEV
            if s < N_DEV - 2:
                comm_ref[s + 1, :, :] = (
                    comm_ref[s + 1, :, :]
                    + partial_ref[pl.ds(c * CHUNK, CHUNK), :]
                )
            else:
                out_ref[:, :] = (
                    comm_ref[s + 1, :, :]
                    + partial_ref[pl.ds(my * CHUNK, CHUNK), :]
                    + shared
                )

    return pl.pallas_call(
        body,
        out_shape=jax.ShapeDtypeStruct((CHUNK, D_HID), jnp.float32),
        in_specs=[pl.BlockSpec(memory_space=pltpu.VMEM)] * 5,
        out_specs=pl.BlockSpec(memory_space=pltpu.VMEM),
        scratch_shapes=[
            pltpu.VMEM((N_TOKENS, D_HID), jnp.float32),
            pltpu.VMEM((N_DEV, CHUNK, D_HID), jnp.float32),
            pltpu.SemaphoreType.DMA((N_DEV - 1,)),
            pltpu.SemaphoreType.DMA((N_DEV - 1,)),
        ],
        compiler_params=pltpu.CompilerParams(collective_id=0),
    )(x, router_W, route_idx, expert_W, shared_W)


# device time: 24431 ns/iter; 1.3996x vs baseline; 1.3996x over previous
import jax
import jax.numpy as jnp
from jax import lax
from jax.experimental import pallas as pl
from jax.experimental.pallas import tpu as pltpu

N_DEV = 4
N_LOCAL_EXPERTS = 4
N_TOKENS = 1024
D_MODEL = 256
D_HID = 512
CHUNK = N_TOKENS // N_DEV


def kernel(x, router_W, route_idx, expert_W, shared_W):
    def body(x_ref, router_ref, idx_ref, ew_ref, sw_ref, out_ref,
             send_buf, comm_ref, send_sems, recv_sems):
        my = lax.axis_index("i")

        barrier_sem = pltpu.get_barrier_semaphore()
        for k in range(1, N_DEV):
            pl.semaphore_signal(
                barrier_sem, inc=1,
                device_id=((my + k) % N_DEV,),
                device_id_type=pl.DeviceIdType.MESH,
            )
        pl.semaphore_wait(barrier_sem, N_DEV - 1)

        ew_flat = ew_ref[:, :, :].reshape(
            N_LOCAL_EXPERTS * D_MODEL, D_HID)

        def chunk_contrib(t):
            xc = x_ref[pl.ds(t * CHUNK, CHUNK), :]
            idx = idx_ref[pl.ds(t * CHUNK, CHUNK), :]
            scores = jnp.dot(xc, router_ref[:, :],
                             preferred_element_type=jnp.float32)
            s_max = jnp.max(scores, axis=-1, keepdims=True)
            e_s = jnp.exp(scores - s_max)
            probs = e_s / jnp.sum(e_s, axis=-1, keepdims=True)
            col = lax.broadcasted_iota(jnp.int32, scores.shape, 1)
            gate = jnp.sum(jnp.where(col == idx, probs, 0.0), axis=-1,
                           keepdims=True)
            xm = jnp.concatenate(
                [jnp.where(idx == my * N_LOCAL_EXPERTS + e, gate, 0.0) * xc
                 for e in range(N_LOCAL_EXPERTS)],
                axis=1)
            return jnp.dot(xm, ew_flat,
                           preferred_element_type=jnp.float32)

        rdmas = []
        for k in (2, 1, 3):
            t = (my + k) % N_DEV
            send_buf[k - 1, :, :] = chunk_contrib(t)
            rdma = pltpu.make_async_remote_copy(
                src_ref=send_buf.at[k - 1],
                dst_ref=comm_ref.at[3 - k],
                send_sem=send_sems.at[k - 1],
                recv_sem=recv_sems.at[3 - k],
                device_id=(t,),
                device_id_type=pl.DeviceIdType.MESH,
            )
            rdma.start()
            rdmas.append(rdma)

        own = chunk_contrib(my)
        x_own = x_ref[pl.ds(my * CHUNK, CHUNK), :]
        shared = jnp.dot(x_own, sw_ref[:, :],
                         preferred_element_type=jnp.float32)

        for j in range(N_DEV - 1):
            recv = pltpu.make_async_remote_copy(
                src_ref=send_buf.at[0],
                dst_ref=comm_ref.at[j],
                send_sem=send_sems.at[0],
                recv_sem=recv_sems.at[j],
                device_id=(my,),
                device_id_type=pl.DeviceIdType.MESH,
            )
            recv.wait_recv()

        out_ref[:, :] = (
            own + shared
            + comm_ref[0, :, :] + comm_ref[1, :, :] + comm_ref[2, :, :]
        )

        for rdma in rdmas:
            rdma.wait_send()

    return pl.pallas_call(
        body,
        out_shape=jax.ShapeDtypeStruct((CHUNK, D_HID), jnp.float32),
        in_specs=[pl.BlockSpec(memory_space=pltpu.VMEM)] * 5,
        out_specs=pl.BlockSpec(memory_space=pltpu.VMEM),
        scratch_shapes=[
            pltpu.VMEM((N_DEV - 1, CHUNK, D_HID), jnp.float32),
            pltpu.VMEM((N_DEV - 1, CHUNK, D_HID), jnp.float32),
            pltpu.SemaphoreType.DMA((N_DEV - 1,)),
            pltpu.SemaphoreType.DMA((N_DEV - 1,)),
        ],
        compiler_params=pltpu.CompilerParams(collective_id=0),
    )(x, router_W, route_idx, expert_W, shared_W)


# device time: 18762 ns/iter; 1.8225x vs baseline; 1.3022x over previous
import jax
import jax.numpy as jnp
from jax import lax
from jax.experimental import pallas as pl
from jax.experimental.pallas import tpu as pltpu

N_DEV = 4
N_LOCAL_EXPERTS = 4
N_TOKENS = 1024
D_MODEL = 256
D_HID = 512
CHUNK = N_TOKENS // N_DEV


def kernel(x, router_W, route_idx, expert_W, shared_W):
    def body(x_ref, router_ref, idx_ref, ew_ref, sw_ref, out_ref,
             ew_bf, send_buf, comm_ref, send_sems, recv_sems):
        my = lax.axis_index("i")

        barrier_sem = pltpu.get_barrier_semaphore()
        for k in range(1, N_DEV):
            pl.semaphore_signal(
                barrier_sem, inc=1,
                device_id=((my + k) % N_DEV,),
                device_id_type=pl.DeviceIdType.MESH,
            )
        pl.semaphore_wait(barrier_sem, N_DEV - 1)

        ew_bf[:, :] = ew_ref[:, :, :].reshape(
            N_LOCAL_EXPERTS * D_MODEL, D_HID).astype(jnp.bfloat16)

        def chunk_contrib(t):
            xc = x_ref[pl.ds(t * CHUNK, CHUNK), :]
            idx = idx_ref[pl.ds(t * CHUNK, CHUNK), :]
            scores = jnp.dot(xc, router_ref[:, :],
                             preferred_element_type=jnp.float32)
            s_max = jnp.max(scores, axis=-1, keepdims=True)
            e_s = jnp.exp(scores - s_max)
            probs = e_s / jnp.sum(e_s, axis=-1, keepdims=True)
            col = lax.broadcasted_iota(jnp.int32, scores.shape, 1)
            gate = jnp.sum(jnp.where(col == idx, probs, 0.0), axis=-1,
                           keepdims=True)
            xm = jnp.concatenate(
                [jnp.where(idx == my * N_LOCAL_EXPERTS + e, gate, 0.0) * xc
                 for e in range(N_LOCAL_EXPERTS)],
                axis=1).astype(jnp.bfloat16)
            return jnp.dot(xm, ew_bf[:, :],
                           preferred_element_type=jnp.float32)

        rdmas = []
        for k in (2, 1, 3):
            t = (my + k) % N_DEV
            send_buf[k - 1, :, :] = chunk_contrib(t).astype(jnp.bfloat16)
            rdma = pltpu.make_async_remote_copy(
                src_ref=send_buf.at[k - 1],
                dst_ref=comm_ref.at[3 - k],
                send_sem=send_sems.at[k - 1],
                recv_sem=recv_sems.at[3 - k],
                device_id=(t,),
                device_id_type=pl.DeviceIdType.MESH,
            )
            rdma.start()
            rdmas.append(rdma)

        own = chunk_contrib(my)
        x_own = x_ref[pl.ds(my * CHUNK, CHUNK), :].astype(jnp.bfloat16)
        shared = jnp.dot(x_own, sw_ref[:, :].astype(jnp.bfloat16),
                         preferred_element_type=jnp.float32)

        for j in range(N_DEV - 1):
            recv = pltpu.make_async_remote_copy(
                src_ref=send_buf.at[0],
                dst_ref=comm_ref.at[j],
                send_sem=send_sems.at[0],
                recv_sem=recv_sems.at[j],
                device_id=(my,),
                device_id_type=pl.DeviceIdType.MESH,
            )
            recv.wait_recv()

        out_ref[:, :] = (
            own + shared
            + comm_ref[0, :, :].astype(jnp.float32)
            + comm_ref[1, :, :].astype(jnp.float32)
            + comm_ref[2, :, :].astype(jnp.float32)
        )

        for rdma in rdmas:
            rdma.wait_send()

    return pl.pallas_call(
        body,
        out_shape=jax.ShapeDtypeStruct((CHUNK, D_HID), jnp.float32),
        in_specs=[pl.BlockSpec(memory_space=pltpu.VMEM)] * 5,
        out_specs=pl.BlockSpec(memory_space=pltpu.VMEM),
        scratch_shapes=[
            pltpu.VMEM((N_LOCAL_EXPERTS * D_MODEL, D_HID),
                       jnp.bfloat16),
            pltpu.VMEM((N_DEV - 1, CHUNK, D_HID), jnp.bfloat16),
            pltpu.VMEM((N_DEV - 1, CHUNK, D_HID), jnp.bfloat16),
            pltpu.SemaphoreType.DMA((N_DEV - 1,)),
            pltpu.SemaphoreType.DMA((N_DEV - 1,)),
        ],
        compiler_params=pltpu.CompilerParams(collective_id=0),
    )(x, router_W, route_idx, expert_W, shared_W)
